# baseline (speedup 1.0000x reference)
import functools

import jax
import jax.numpy as jnp
from jax import lax
from jax.experimental import pallas as pl
from jax.experimental.pallas import tpu as pltpu
from jax.experimental.pallas import tpu_sc as plsc

_LANE = 128
_BPT = 256
_RING = 4
_DUMP_R = -(1 << 20)


@functools.lru_cache(maxsize=None)
def _make_gather(V, D, B):
    info = plsc.get_sparse_core_info()
    NC, NS = info.num_cores, info.num_subcores
    NW = NC * NS
    n_idx_vregs = B // 16
    max_col = ((V - 1) >> 7) * _LANE
    b1_sz = B + 16 * _BPT + 32
    b2_sz = B + 16 * 16 + 32
    mesh = plsc.VectorSubcoreMesh(core_axis_name="c", subcore_axis_name="s")

    @functools.partial(
        pl.kernel,
        mesh=mesh,
        out_type=jax.ShapeDtypeStruct((B + 8, _LANE), jnp.float32),
        scratch_types=[
            pltpu.VMEM((b1_sz,), jnp.int32),
            pltpu.VMEM((b1_sz,), jnp.int32),
            pltpu.VMEM((b2_sz,), jnp.int32),
            pltpu.VMEM((b2_sz,), jnp.int32),
            pltpu.VMEM((_BPT + 16,), jnp.int32),
            pltpu.VMEM((32,), jnp.int32),
            pltpu.VMEM((_RING * D, _LANE), jnp.float32),
            pltpu.VMEM((64, _LANE), jnp.float32),
            pltpu.VMEM((2, 32), jnp.int32),
        ] + [pltpu.SemaphoreType.DMA] * _RING,
        compiler_params=pltpu.CompilerParams(
            use_tc_tiling_on_sc=True, needs_layout_passes=False
        ),
    )
    def gather_kernel(table_hbm, idx_hbm, out_hbm,
                      b1_r, b1_b, b2_r, b2_b, meta, bsv, blk_v, stage_v,
                      bst_v, *sems):
        wid = lax.axis_index("s") * NC + lax.axis_index("c")
        lo_blk = wid * _BPT
        vlo = lo_blk * _LANE
        vhi = vlo + _BPT * _LANE
        iota = lax.iota(jnp.int32, 16)
        lane0 = iota == 0
        dump_r_v = jnp.full((16,), _DUMP_R, jnp.int32)
        dump_b_v = jnp.full((16,), B, jnp.int32)

        def popcnt(m):
            return plsc.all_reduce_population_count(m)[0]

        def pf1(i, _):
            b1_r[pl.ds(i * 16, 16)] = dump_r_v
            b1_b[pl.ds(i * 16, 16)] = dump_b_v
            return 0

        lax.fori_loop(0, b1_sz // 16, pf1, 0)
        bst_v[0, pl.ds(0, 16)] = dump_b_v
        bst_v[0, pl.ds(16, 16)] = dump_b_v
        bst_v[1, pl.ds(0, 16)] = dump_b_v
        bst_v[1, pl.ds(16, 16)] = dump_b_v

        pltpu.sync_copy(idx_hbm, b2_r.at[pl.ds(0, B)])

        def p0(i, cnt):
            r = b2_r[pl.ds(i * 16, 16)]
            bp = jnp.full((16,), i * 16, jnp.int32) + iota
            m = (r >= vlo) & (r < vhi)
            plsc.store_compressed(b1_r.at[pl.ds(cnt, 16)], r, mask=m)
            plsc.store_compressed(b1_b.at[pl.ds(cnt, 16)], bp, mask=m)
            return cnt + popcnt(m)

        own_n = lax.fori_loop(0, n_idx_vregs, p0, 0)
        nv0 = (own_n + 15) >> 4

        def pf2(i, _):
            b2_r[pl.ds(i * 16, 16)] = dump_r_v
            b2_b[pl.ds(i * 16, 16)] = dump_b_v
            return 0

        lax.fori_loop(0, b2_sz // 16, pf2, 0)
        plsc.store_scatter(bsv, [jnp.zeros((16,), jnp.int32)],
                           jnp.zeros((16,), jnp.int32), mask=lane0)
        cnt1 = 0
        for k in range(16):
            def l1(i, cnt, k=k):
                r = b1_r[pl.ds(i * 16, 16)]
                bp = b1_b[pl.ds(i * 16, 16)]
                m = (((r >> 7) - lo_blk) >> 4) == k
                plsc.store_compressed(b2_r.at[pl.ds(cnt, 16)], r, mask=m)
                plsc.store_compressed(b2_b.at[pl.ds(cnt, 16)], bp, mask=m)
                return cnt + popcnt(m)

            cnt1 = lax.fori_loop(0, nv0, l1, cnt1)
            cnt1 = (cnt1 + 15) & ~15
            plsc.store_scatter(bsv, [jnp.full((16,), k + 1, jnp.int32)],
                               jnp.full((16,), cnt1, jnp.int32), mask=lane0)

        lax.fori_loop(0, b1_sz // 16, pf1, 0)

        def l2(j, cnt):
            k = j >> 4
            kb0 = plsc.load_gather(bsv, [jnp.full((16,), k, jnp.int32)])[0]
            kb1 = plsc.load_gather(bsv, [jnp.full((16,), k + 1, jnp.int32)])[0]
            plsc.store_scatter(meta, [jnp.full((16,), j, jnp.int32)],
                               jnp.full((16,), cnt, jnp.int32), mask=lane0)

            def body(i, cnt):
                r = b2_r[pl.ds(kb0 + i * 16, 16)]
                bp = b2_b[pl.ds(kb0 + i * 16, 16)]
                m = ((r >> 7) - lo_blk) == j
                plsc.store_compressed(b1_r.at[pl.ds(cnt, 16)], r, mask=m)
                plsc.store_compressed(b1_b.at[pl.ds(cnt, 16)], bp, mask=m)
                return cnt + popcnt(m)

            cnt = lax.fori_loop(0, (kb1 - kb0) >> 4, body, cnt)
            return (cnt + 15) & ~15

        cnt2 = lax.fori_loop(0, _BPT, l2, 0)
        plsc.store_scatter(meta, [jnp.full((16,), _BPT, jnp.int32)],
                           jnp.full((16,), cnt2, jnp.int32), mask=lane0)

        def fire(j, s):
            col = jnp.clip((lo_blk + j) * _LANE, 0, max_col)
            col = pl.multiple_of(col, _LANE)
            return pltpu.async_copy(
                table_hbm.at[:, pl.ds(col, _LANE)],
                blk_v.at[pl.ds(s * D, D)], sems[s],
            )

        def slot_wait(s):
            pltpu.make_async_copy(
                table_hbm.at[:, pl.ds(0, _LANE)],
                blk_v.at[pl.ds(s * D, D)], sems[s],
            ).wait()

        def flush(cnt_after):
            half = ((cnt_after - 1) >> 5) & 1
            src = stage_v.at[pl.ds(half * 32, 32)]
            pltpu.sync_copy(src, out_hbm.at[bst_v.at[half]])
            bst_v[half, pl.ds(0, 16)] = dump_b_v
            bst_v[half, pl.ds(16, 16)] = dump_b_v

        for s in range(_RING):
            fire(s, s)

        def sweep_g(g, cnt):
            for s in range(_RING):
                j = g * _RING + s
                mv = meta[pl.ds(j, 16)]
                start, end = mv[0], mv[1]
                slot_wait(s)

                def ext(q, cnt, s=s):
                    r_v = b1_r[pl.ds(start + q * 16, 16)]
                    b_v = b1_b[pl.ds(start + q * 16, 16)]
                    half = (cnt >> 5) & 1
                    for l in range(16):
                        r = r_v[l]
                        row = jnp.full((16,), (cnt + l) & 31, jnp.int32)
                        lane_v = jnp.full((16,), r & (_LANE - 1), jnp.int32)
                        plsc.store_scatter(
                            bst_v.at[half],
                            [jnp.full((16,), (cnt + l) & 31, jnp.int32)],
                            jnp.full((16,), b_v[l], jnp.int32), mask=lane0)
                        for q2 in range(D // 16):
                            f_v = iota + (s * D + q2 * 16)
                            vals = plsc.load_gather(blk_v, [f_v, lane_v])
                            plsc.store_scatter(
                                stage_v,
                                [half * 32 + row, iota + q2 * 16], vals)
                    cnt = cnt + 16

                    @pl.when((cnt & 31) == 0)
                    def _():
                        flush(cnt)

                    return cnt

                cnt = lax.fori_loop(0, (end - start) >> 4, ext, cnt)
                fire(j + _RING, s)
            return cnt

        cnt_end = lax.fori_loop(0, _BPT // _RING, sweep_g, 0)

        @pl.when((cnt_end & 31) != 0)
        def _():
            flush(cnt_end + 1)

        for s in range(_RING):
            slot_wait(s)

    return gather_kernel


def kernel(inputs, embedding):
    B = inputs.shape[0]
    V = embedding.shape[0]
    D = embedding.shape[-1]
    table_t = embedding.reshape(V, D).T
    idx = inputs.astype(jnp.int32)
    out_w = _make_gather(V, D, B)(table_t, idx)
    return out_w[:B, :D].reshape(inputs.shape + (1, 1, D))


# speedup vs baseline: 18.5201x; 18.5201x over previous
import functools

import jax
import jax.numpy as jnp
from jax import lax
from jax.experimental import pallas as pl
from jax.experimental.pallas import tpu as pltpu
from jax.experimental.pallas import tpu_sc as plsc

_LANE = 128
_BLK = 128
_NBUF = 8


@functools.lru_cache(maxsize=None)
def _make_gather(V, D, B):
    info = plsc.get_sparse_core_info()
    NC, NS = info.num_cores, info.num_subcores
    NW = NC * NS
    per_w = B // NW
    n_ch = per_w // 16
    flushes = _BLK // 16
    max_col = ((V - 1) >> 7) * _LANE
    mesh = plsc.VectorSubcoreMesh(core_axis_name="c", subcore_axis_name="s")

    @functools.partial(
        pl.kernel,
        mesh=mesh,
        out_type=jax.ShapeDtypeStruct((D, B), jnp.float32),
        scratch_types=[
            pltpu.VMEM((per_w + 16,), jnp.int32),
            pltpu.VMEM((_NBUF, D, _LANE), jnp.float32),
            pltpu.VMEM((D, _BLK), jnp.float32),
        ] + [pltpu.SemaphoreType.DMA] * _NBUF,
        compiler_params=pltpu.CompilerParams(
            use_tc_tiling_on_sc=True, needs_layout_passes=False
        ),
    )
    def gather_kernel(table_hbm, idx_hbm, out_hbm, idx_v, blk_v, stage_v,
                      *sems):
        wid = lax.axis_index("s") * NC + lax.axis_index("c")
        b0 = wid * per_w
        pltpu.sync_copy(idx_hbm.at[pl.ds(b0, per_w)], idx_v.at[pl.ds(0, per_w)])
        row_ids = lax.iota(jnp.int32, 16)

        def fire(r, slot):
            col = jnp.clip((r >> 7) * _LANE, 0, max_col)
            col = pl.multiple_of(col, _LANE)
            return pltpu.async_copy(
                table_hbm.at[:, pl.ds(col, _LANE)], blk_v.at[slot], sems[slot]
            )

        def slot_wait(slot):
            pltpu.make_async_copy(
                table_hbm.at[:, pl.ds(0, _LANE)], blk_v.at[slot], sems[slot]
            ).wait()

        def extract(r, k, slot):
            lane_v = jnp.full((16,), r & (_LANE - 1), jnp.int32)
            k_v = jnp.full((16,), k, jnp.int32)
            for q in range(D // 16):
                f_v = row_ids + q * 16
                vals = plsc.load_gather(blk_v.at[slot], [f_v, lane_v])
                plsc.store_scatter(stage_v, [f_v, k_v], vals)

        r0 = idx_v[pl.ds(0, 16)]
        for j in range(_NBUF):
            fire(r0[j], j)

        def do_chunk(ch, _):
            base = ch * 16
            r_cur = idx_v[pl.ds(base, 16)]
            r_nxt = idx_v[pl.ds(base + 16, 16)]
            kc = (ch % flushes) * 16
            for j in range(16):
                slot = j % _NBUF
                slot_wait(slot)
                extract(r_cur[j], kc + j, slot)
                if j < _NBUF:
                    fire(r_cur[j + _NBUF], slot)
                else:
                    fire(r_nxt[j - _NBUF], slot)

            @pl.when(ch % flushes == flushes - 1)
            def _flush():
                start = pl.multiple_of(b0 + (ch + 1) * 16 - _BLK, _BLK)
                pltpu.sync_copy(stage_v, out_hbm.at[:, pl.ds(start, _BLK)])

            return 0

        lax.fori_loop(0, n_ch, do_chunk, 0)
        for j in range(_NBUF):
            slot_wait(j)

    return gather_kernel


def kernel(inputs, embedding):
    B = inputs.shape[0]
    V = embedding.shape[0]
    D = embedding.shape[-1]
    table_t = embedding.reshape(V, D).T
    idx = inputs.astype(jnp.int32)
    out_t = _make_gather(V, D, B)(table_t, idx)
    return out_t.T.reshape(inputs.shape + (1, 1, D))
